# Initial kernel scaffold; baseline (speedup 1.0000x reference)
import jax
import jax.numpy as jnp
from jax.experimental import pallas as pl


def kernel(node_subgraph, edge_index, edge_weight, feat_full, label_full, W0a, b0a, W1a, b1a, W0b, b0b, W1b, b1b, Wc, bc):
    raise NotImplementedError("write your pallas kernel here")


# speedup vs baseline: 2.3897x; 2.3897x over previous
import functools

import jax
import jax.numpy as jnp
from jax import lax
from jax.experimental import pallas as pl
from jax.experimental.pallas import tpu as pltpu
from jax.experimental.pallas import tpu_sc as plsc

NC = 2
NS = 16
L = 16
NW = NC * NS

N_SUB = 10000
R_PAD = 10240
D = 128
E = 320000
E_PAD = 327680
EPT = E_PAD // NW
K = 128
NCHUNK = EPT // K
BPT = R_PAD // NW
GJ = 4
GK = BPT // GJ
RPT = R_PAD // NS
N_FULL = 50000
C_CLS = 41
BR = 1280
GRID = R_PAD // BR

_vmesh = plsc.VectorSubcoreMesh(
    core_axis_name="c", subcore_axis_name="s", num_cores=NC, num_subcores=NS)


def _gather_body(feat_hbm, cat_hbm, idx_hbm, feat_out, cat_out,
                 idx_v, frows, gout, sem):
  c = lax.axis_index("c")
  s = lax.axis_index("s")
  wid = s * NC + c
  base = wid * BPT
  pltpu.sync_copy(idx_hbm.at[wid], idx_v)
  for j in range(GJ):
    pltpu.async_copy(feat_hbm.at[idx_v.at[j]], frows, sem).wait()
    pltpu.sync_copy(frows, feat_out.at[pl.ds(base + j * GK, GK)])
    pltpu.async_copy(cat_hbm.at[idx_v.at[j]], gout.at[j], sem).wait()
    pltpu.sync_copy(gout.at[j], cat_out.at[pl.ds(base + j * GK, GK)])


@jax.jit
def _sc_gather(feat_full, cat_full, idx_pad):
  grid = idx_pad.reshape(NW, GJ, GK)
  return pl.kernel(
      _gather_body,
      out_type=(jax.ShapeDtypeStruct((R_PAD, D), jnp.float32),
                jax.ShapeDtypeStruct((R_PAD,), jnp.int32)),
      mesh=_vmesh,
      scratch_types=[
          pltpu.VMEM((GJ, GK), jnp.int32),
          pltpu.VMEM((GK, D), jnp.float32),
          pltpu.VMEM((GJ, GK), jnp.int32),
          pltpu.SemaphoreType.DMA,
      ],
  )(feat_full, cat_full, grid)


def _tc_cat_body(lab_ref, cat_ref):
  lab = lab_ref[...]
  mx = jnp.max(lab, axis=1, keepdims=True)
  io = lax.broadcasted_iota(jnp.int32, lab.shape, 1)
  cat_ref[...] = jnp.min(jnp.where(lab == mx, io, 64), axis=1, keepdims=True)


@jax.jit
def _tc_cat(label_full):
  return pl.pallas_call(
      _tc_cat_body,
      grid=(N_FULL // 2000,),
      in_specs=[pl.BlockSpec((2000, 41), lambda i: (i, 0))],
      out_specs=pl.BlockSpec((2000, 1), lambda i: (i, 0)),
      out_shape=jax.ShapeDtypeStruct((N_FULL, 1), jnp.int32),
  )(label_full)


GC = 8
NG = NCHUNK // GC


def _spmm_body(x_hbm, src_hbm, dst_hbm, w_hbm, out_hbm,
               src_v, dst_v, w_v, rows_v, zbuf, acc_sh, sem):
  c = lax.axis_index("c")
  s = lax.axis_index("s")
  wid = s * NC + c

  zero = jnp.zeros((L,), jnp.float32)

  @pl.loop(0, K)
  def _zrow(r):
    for cc in range(D // L):
      zbuf[r, pl.ds(cc * L, L)] = zero

  for t in range(RPT // K):
    pltpu.sync_copy(zbuf, acc_sh.at[pl.ds(s * RPT + t * K, K)])
  plsc.subcore_barrier()

  @pl.loop(0, NG)
  def _group(g):
    gsl = pl.ds(g * GC, GC)
    pltpu.sync_copy(src_hbm.at[wid, gsl], src_v)
    pltpu.sync_copy(dst_hbm.at[wid, gsl], dst_v)
    pltpu.sync_copy(w_hbm.at[wid, gsl], w_v)

    for j in range(GC):
      pltpu.async_copy(x_hbm.at[src_v.at[j]], rows_v, sem).wait()

      @pl.loop(0, K // L)
      def _grp(gg):
        wvec = w_v[j, pl.ds(gg * L, L)]
        for i in range(L):
          wv = wvec[i]
          r = gg * L + i
          for cc in range(D // L):
            sl = pl.ds(cc * L, L)
            rows_v[r, sl] = rows_v[r, sl] * wv

      pltpu.sync_copy(rows_v, acc_sh.at[dst_v.at[j]], add=True)

  plsc.subcore_barrier()
  pltpu.sync_copy(acc_sh.at[pl.ds(s * RPT, RPT)],
                  out_hbm.at[c, pl.ds(s * RPT, RPT)])


@jax.jit
def _sc_spmm(x, src_g, dst_g, w_g):
  return pl.kernel(
      _spmm_body,
      out_type=jax.ShapeDtypeStruct((NC, R_PAD, D), jnp.float32),
      mesh=_vmesh,
      scratch_types=[
          pltpu.VMEM((GC, K), jnp.int32),
          pltpu.VMEM((GC, K), jnp.int32),
          pltpu.VMEM((GC, K), jnp.float32),
          pltpu.VMEM((K, D), jnp.float32),
          pltpu.VMEM((K, D), jnp.float32),
          pltpu.VMEM_SHARED((R_PAD, D), jnp.float32),
          pltpu.SemaphoreType.DMA,
      ],
  )(x, src_g, dst_g, w_g)


def _tc_a_body(x_ref, sp_ref, w0_ref, b0_ref, w1_ref, b1_ref, out_ref):
  x = x_ref[...]
  sp = sp_ref[0] + sp_ref[1]
  h0 = jnp.maximum(
      jnp.dot(x, w0_ref[...], preferred_element_type=jnp.float32)
      + b0_ref[...], 0.0)
  h1 = jnp.maximum(
      jnp.dot(sp, w1_ref[...], preferred_element_type=jnp.float32)
      + b1_ref[...], 0.0)
  out_ref[...] = jnp.concatenate([h0, h1], axis=1)


@jax.jit
def _tc_layer1(x, sp, w0, b0, w1, b1):
  return pl.pallas_call(
      _tc_a_body,
      grid=(GRID,),
      in_specs=[
          pl.BlockSpec((BR, D), lambda i: (i, 0)),
          pl.BlockSpec((NC, BR, D), lambda i: (0, i, 0)),
          pl.BlockSpec((D, D), lambda i: (0, 0)),
          pl.BlockSpec((1, D), lambda i: (0, 0)),
          pl.BlockSpec((D, D), lambda i: (0, 0)),
          pl.BlockSpec((1, D), lambda i: (0, 0)),
      ],
      out_specs=pl.BlockSpec((BR, 2 * D), lambda i: (i, 0)),
      out_shape=jax.ShapeDtypeStruct((R_PAD, 2 * D), jnp.float32),
  )(x, sp, w0, b0.reshape(1, D), w1, b1.reshape(1, D))


def _tc_b_body(h_ref, spl_ref, spr_ref, w0_ref, b0_ref, w1_ref, b1_ref,
               wc_ref, bc_ref, cat_ref, pred_ref, lab_ref):
  h = h_ref[...]
  sp = jnp.concatenate([spl_ref[0] + spl_ref[1], spr_ref[0] + spr_ref[1]],
                       axis=1)
  h0 = jnp.maximum(
      jnp.dot(h, w0_ref[...], preferred_element_type=jnp.float32)
      + b0_ref[...], 0.0)
  h1 = jnp.maximum(
      jnp.dot(sp, w1_ref[...], preferred_element_type=jnp.float32)
      + b1_ref[...], 0.0)
  emb = jnp.concatenate([h0, h1], axis=1)
  n2 = jnp.sum(emb * emb, axis=1, keepdims=True)
  embn = emb / jnp.maximum(jnp.sqrt(n2), 1e-12)
  pred_ref[...] = (
      jnp.dot(embn, wc_ref[...], preferred_element_type=jnp.float32)
      + bc_ref[...])
  io = lax.broadcasted_iota(jnp.int32, (BR, C_CLS), 1)
  lab_ref[...] = (cat_ref[...] == io).astype(jnp.float32)


@jax.jit
def _tc_layer2(h, spl, spr, w0, b0, w1, b1, wc, bc, cat):
  return pl.pallas_call(
      _tc_b_body,
      grid=(GRID,),
      in_specs=[
          pl.BlockSpec((BR, 2 * D), lambda i: (i, 0)),
          pl.BlockSpec((NC, BR, D), lambda i: (0, i, 0)),
          pl.BlockSpec((NC, BR, D), lambda i: (0, i, 0)),
          pl.BlockSpec((2 * D, D), lambda i: (0, 0)),
          pl.BlockSpec((1, D), lambda i: (0, 0)),
          pl.BlockSpec((2 * D, D), lambda i: (0, 0)),
          pl.BlockSpec((1, D), lambda i: (0, 0)),
          pl.BlockSpec((2 * D, C_CLS), lambda i: (0, 0)),
          pl.BlockSpec((1, C_CLS), lambda i: (0, 0)),
          pl.BlockSpec((BR, 1), lambda i: (i, 0)),
      ],
      out_specs=[
          pl.BlockSpec((BR, C_CLS), lambda i: (i, 0)),
          pl.BlockSpec((BR, C_CLS), lambda i: (i, 0)),
      ],
      out_shape=[
          jax.ShapeDtypeStruct((R_PAD, C_CLS), jnp.float32),
          jax.ShapeDtypeStruct((R_PAD, C_CLS), jnp.float32),
      ],
  )(h, spl, spr, w0, b0.reshape(1, D), w1, b1.reshape(1, D),
    wc, bc.reshape(1, C_CLS), cat)


def kernel(node_subgraph, edge_index, edge_weight, feat_full, label_full,
           W0a, b0a, W1a, b1a, W0b, b0b, W1b, b1b, Wc, bc):
  idx_pad = jnp.pad(node_subgraph, (0, R_PAD - N_SUB))
  src = jnp.pad(edge_index[0], (0, E_PAD - E)).reshape(NW, NCHUNK, K)
  dst = jnp.pad(edge_index[1], (0, E_PAD - E)).reshape(NW, NCHUNK, K)
  w = jnp.pad(edge_weight, (0, E_PAD - E)).reshape(NW, NCHUNK, K)

  cat_full = _tc_cat(label_full).reshape(N_FULL)
  feat_subg, cat_sub = _sc_gather(feat_full, cat_full, idx_pad)

  sp1 = _sc_spmm(feat_subg, src, dst, w)
  h = _tc_layer1(feat_subg, sp1, W0a, b0a, W1a, b1a)

  sp2l = _sc_spmm(h[:, :D], src, dst, w)
  sp2r = _sc_spmm(h[:, D:], src, dst, w)
  pred_pad, lab_pad = _tc_layer2(h, sp2l, sp2r, W0b, b0b, W1b, b1b, Wc, bc,
                                 cat_sub.reshape(R_PAD, 1))

  pred = pred_pad[:N_SUB]
  label_subg = lab_pad[:N_SUB]
  conv = cat_sub[:N_SUB]
  return pred, label_subg, conv


# speedup vs baseline: 3.2739x; 1.3700x over previous
import functools

import jax
import jax.numpy as jnp
from jax import lax
from jax.experimental import pallas as pl
from jax.experimental.pallas import tpu as pltpu
from jax.experimental.pallas import tpu_sc as plsc

NC = 2
NS = 16
L = 16
NW = NC * NS

N_SUB = 10000
R_PAD = 10240
D = 128
E = 320000
E_PAD = 327680
EPT = E_PAD // NW
K = 128
NCHUNK = EPT // K
BPT = R_PAD // NW
GJ = 4
GK = BPT // GJ
RPT = R_PAD // NS
N_FULL = 50000
C_CLS = 41
BR = 1280
GRID = R_PAD // BR

_vmesh = plsc.VectorSubcoreMesh(
    core_axis_name="c", subcore_axis_name="s", num_cores=NC, num_subcores=NS)


def _gather_body(feat_hbm, cat_hbm, idx_hbm, feat_out, cat_out,
                 idx_v, frows, gout, sem):
  c = lax.axis_index("c")
  s = lax.axis_index("s")
  wid = s * NC + c
  base = wid * BPT
  pltpu.sync_copy(idx_hbm.at[wid], idx_v)
  for j in range(GJ):
    pltpu.async_copy(feat_hbm.at[idx_v.at[j]], frows, sem).wait()
    pltpu.sync_copy(frows, feat_out.at[pl.ds(base + j * GK, GK)])
    pltpu.async_copy(cat_hbm.at[idx_v.at[j]], gout.at[j], sem).wait()
    pltpu.sync_copy(gout.at[j], cat_out.at[pl.ds(base + j * GK, GK)])


@jax.jit
def _sc_gather(feat_full, cat_full, idx_pad):
  grid = idx_pad.reshape(NW, GJ, GK)
  return pl.kernel(
      _gather_body,
      out_type=(jax.ShapeDtypeStruct((R_PAD, D), jnp.float32),
                jax.ShapeDtypeStruct((R_PAD,), jnp.int32)),
      mesh=_vmesh,
      scratch_types=[
          pltpu.VMEM((GJ, GK), jnp.int32),
          pltpu.VMEM((GK, D), jnp.float32),
          pltpu.VMEM((GJ, GK), jnp.int32),
          pltpu.SemaphoreType.DMA,
      ],
  )(feat_full, cat_full, grid)


def _tc_cat_body(lab_ref, cat_ref):
  lab = lab_ref[...]
  mx = jnp.max(lab, axis=1, keepdims=True)
  io = lax.broadcasted_iota(jnp.int32, lab.shape, 1)
  cat_ref[...] = jnp.min(jnp.where(lab == mx, io, 64), axis=1, keepdims=True)


@jax.jit
def _tc_cat(label_full):
  return pl.pallas_call(
      _tc_cat_body,
      grid=(N_FULL // 2000,),
      in_specs=[pl.BlockSpec((2000, 41), lambda i: (i, 0))],
      out_specs=pl.BlockSpec((2000, 1), lambda i: (i, 0)),
      out_shape=jax.ShapeDtypeStruct((N_FULL, 1), jnp.int32),
  )(label_full)


GC = 8
NG = NCHUNK // GC


def _spmm_body(x_hbm, src_hbm, dst_hbm, w_hbm, out_hbm,
               src_v, dst_v, w_v, rows_v, zbuf, acc_sh, sem):
  c = lax.axis_index("c")
  s = lax.axis_index("s")
  wid = s * NC + c

  zero = jnp.zeros((L,), jnp.float32)

  @pl.loop(0, K)
  def _zrow(r):
    for cc in range(D // L):
      zbuf[r, pl.ds(cc * L, L)] = zero

  for t in range(RPT // K):
    pltpu.sync_copy(zbuf, acc_sh.at[pl.ds(s * RPT + t * K, K)])
  plsc.subcore_barrier()

  @pl.loop(0, NG)
  def _group(g):
    gsl = pl.ds(g * GC, GC)
    pltpu.sync_copy(src_hbm.at[wid, gsl], src_v)
    pltpu.sync_copy(dst_hbm.at[wid, gsl], dst_v)
    pltpu.sync_copy(w_hbm.at[wid, gsl], w_v)

    for j in range(GC):
      pltpu.async_copy(x_hbm.at[src_v.at[j]], rows_v, sem).wait()

      @pl.loop(0, K // L)
      def _grp(gg):
        wvec = w_v[j, pl.ds(gg * L, L)]
        for i in range(L):
          wv = wvec[i]
          r = gg * L + i
          for cc in range(D // L):
            sl = pl.ds(cc * L, L)
            rows_v[r, sl] = rows_v[r, sl] * wv

      pltpu.sync_copy(rows_v, acc_sh.at[dst_v.at[j]], add=True)

  plsc.subcore_barrier()
  pltpu.sync_copy(acc_sh.at[pl.ds(s * RPT, RPT)],
                  out_hbm.at[c, pl.ds(s * RPT, RPT)])


@jax.jit
def _sc_spmm(x, src_g, dst_g, w_g):
  return pl.kernel(
      _spmm_body,
      out_type=jax.ShapeDtypeStruct((NC, R_PAD, D), jnp.float32),
      mesh=_vmesh,
      scratch_types=[
          pltpu.VMEM((GC, K), jnp.int32),
          pltpu.VMEM((GC, K), jnp.int32),
          pltpu.VMEM((GC, K), jnp.float32),
          pltpu.VMEM((K, D), jnp.float32),
          pltpu.VMEM((K, D), jnp.float32),
          pltpu.VMEM_SHARED((R_PAD, D), jnp.float32),
          pltpu.SemaphoreType.DMA,
      ],
  )(x, src_g, dst_g, w_g)


def _tc_a_body(x_ref, sp_ref, w0_ref, b0_ref, w1_ref, b1_ref, w1b_ref,
               h_ref, y_ref):
  x = x_ref[...]
  sp = sp_ref[0] + sp_ref[1]
  h0 = jnp.maximum(
      jnp.dot(x, w0_ref[...], preferred_element_type=jnp.float32)
      + b0_ref[...], 0.0)
  h1 = jnp.maximum(
      jnp.dot(sp, w1_ref[...], preferred_element_type=jnp.float32)
      + b1_ref[...], 0.0)
  h = jnp.concatenate([h0, h1], axis=1)
  h_ref[...] = h
  y_ref[...] = jnp.dot(h, w1b_ref[...], preferred_element_type=jnp.float32)


@jax.jit
def _tc_layer1(x, sp, w0, b0, w1, b1, w1b):
  return pl.pallas_call(
      _tc_a_body,
      grid=(GRID,),
      in_specs=[
          pl.BlockSpec((BR, D), lambda i: (i, 0)),
          pl.BlockSpec((NC, BR, D), lambda i: (0, i, 0)),
          pl.BlockSpec((D, D), lambda i: (0, 0)),
          pl.BlockSpec((1, D), lambda i: (0, 0)),
          pl.BlockSpec((D, D), lambda i: (0, 0)),
          pl.BlockSpec((1, D), lambda i: (0, 0)),
          pl.BlockSpec((2 * D, D), lambda i: (0, 0)),
      ],
      out_specs=[
          pl.BlockSpec((BR, 2 * D), lambda i: (i, 0)),
          pl.BlockSpec((BR, D), lambda i: (i, 0)),
      ],
      out_shape=[
          jax.ShapeDtypeStruct((R_PAD, 2 * D), jnp.float32),
          jax.ShapeDtypeStruct((R_PAD, D), jnp.float32),
      ],
  )(x, sp, w0, b0.reshape(1, D), w1, b1.reshape(1, D), w1b)


def _tc_b_body(h_ref, sp_ref, w0_ref, b0_ref, b1_ref,
               wc_ref, bc_ref, cat_ref, pred_ref, lab_ref):
  h = h_ref[...]
  h0 = jnp.maximum(
      jnp.dot(h, w0_ref[...], preferred_element_type=jnp.float32)
      + b0_ref[...], 0.0)
  h1 = jnp.maximum(sp_ref[0] + sp_ref[1] + b1_ref[...], 0.0)
  emb = jnp.concatenate([h0, h1], axis=1)
  n2 = jnp.sum(emb * emb, axis=1, keepdims=True)
  embn = emb / jnp.maximum(jnp.sqrt(n2), 1e-12)
  pred_ref[...] = (
      jnp.dot(embn, wc_ref[...], preferred_element_type=jnp.float32)
      + bc_ref[...])
  io = lax.broadcasted_iota(jnp.int32, (BR, C_CLS), 1)
  lab_ref[...] = (cat_ref[...] == io).astype(jnp.float32)


@jax.jit
def _tc_layer2(h, sp, w0, b0, b1, wc, bc, cat):
  return pl.pallas_call(
      _tc_b_body,
      grid=(GRID,),
      in_specs=[
          pl.BlockSpec((BR, 2 * D), lambda i: (i, 0)),
          pl.BlockSpec((NC, BR, D), lambda i: (0, i, 0)),
          pl.BlockSpec((2 * D, D), lambda i: (0, 0)),
          pl.BlockSpec((1, D), lambda i: (0, 0)),
          pl.BlockSpec((1, D), lambda i: (0, 0)),
          pl.BlockSpec((2 * D, C_CLS), lambda i: (0, 0)),
          pl.BlockSpec((1, C_CLS), lambda i: (0, 0)),
          pl.BlockSpec((BR, 1), lambda i: (i, 0)),
      ],
      out_specs=[
          pl.BlockSpec((BR, C_CLS), lambda i: (i, 0)),
          pl.BlockSpec((BR, C_CLS), lambda i: (i, 0)),
      ],
      out_shape=[
          jax.ShapeDtypeStruct((R_PAD, C_CLS), jnp.float32),
          jax.ShapeDtypeStruct((R_PAD, C_CLS), jnp.float32),
      ],
  )(h, sp, w0, b0.reshape(1, D), b1.reshape(1, D),
    wc, bc.reshape(1, C_CLS), cat)


def kernel(node_subgraph, edge_index, edge_weight, feat_full, label_full,
           W0a, b0a, W1a, b1a, W0b, b0b, W1b, b1b, Wc, bc):
  idx_pad = jnp.pad(node_subgraph, (0, R_PAD - N_SUB))
  src = jnp.pad(edge_index[0], (0, E_PAD - E)).reshape(NW, NCHUNK, K)
  dst = jnp.pad(edge_index[1], (0, E_PAD - E)).reshape(NW, NCHUNK, K)
  w = jnp.pad(edge_weight, (0, E_PAD - E)).reshape(NW, NCHUNK, K)

  cat_full = _tc_cat(label_full).reshape(N_FULL)
  feat_subg, cat_sub = _sc_gather(feat_full, cat_full, idx_pad)

  sp1 = _sc_spmm(feat_subg, src, dst, w)
  h, y = _tc_layer1(feat_subg, sp1, W0a, b0a, W1a, b1a, W1b)

  sp2 = _sc_spmm(y, src, dst, w)
  pred_pad, lab_pad = _tc_layer2(h, sp2, W0b, b0b, b1b, Wc, bc,
                                 cat_sub.reshape(R_PAD, 1))

  pred = pred_pad[:N_SUB]
  label_subg = lab_pad[:N_SUB]
  conv = cat_sub[:N_SUB]
  return pred, label_subg, conv


# speedup vs baseline: 3.7222x; 1.1369x over previous
import functools

import jax
import jax.numpy as jnp
from jax import lax
from jax.experimental import pallas as pl
from jax.experimental.pallas import tpu as pltpu
from jax.experimental.pallas import tpu_sc as plsc

NC = 2
NS = 16
L = 16
NW = NC * NS

N_SUB = 10000
R_PAD = 10240
D = 128
E = 320000
E_PAD = 327680
EPT = E_PAD // NW
K = 128
NCHUNK = EPT // K
BPT = R_PAD // NW
GJ = 4
GK = BPT // GJ
RPT = R_PAD // NS
N_FULL = 50000
C_CLS = 41
BR = 1280
GRID = R_PAD // BR

_vmesh = plsc.VectorSubcoreMesh(
    core_axis_name="c", subcore_axis_name="s", num_cores=NC, num_subcores=NS)


def _gather_body(feat_hbm, cat_hbm, idx_hbm, feat_out, cat_out,
                 idx_v, frows, gout, sem):
  c = lax.axis_index("c")
  s = lax.axis_index("s")
  wid = s * NC + c
  base = wid * BPT
  pltpu.sync_copy(idx_hbm.at[wid], idx_v)
  for j in range(GJ):
    pltpu.async_copy(feat_hbm.at[idx_v.at[j]], frows, sem).wait()
    pltpu.sync_copy(frows, feat_out.at[pl.ds(base + j * GK, GK)])
    pltpu.async_copy(cat_hbm.at[idx_v.at[j]], gout.at[j], sem).wait()
    pltpu.sync_copy(gout.at[j], cat_out.at[pl.ds(base + j * GK, GK)])


@jax.jit
def _sc_gather(feat_full, cat_full, idx_pad):
  grid = idx_pad.reshape(NW, GJ, GK)
  return pl.kernel(
      _gather_body,
      out_type=(jax.ShapeDtypeStruct((R_PAD, D), jnp.float32),
                jax.ShapeDtypeStruct((R_PAD,), jnp.int32)),
      mesh=_vmesh,
      scratch_types=[
          pltpu.VMEM((GJ, GK), jnp.int32),
          pltpu.VMEM((GK, D), jnp.float32),
          pltpu.VMEM((GJ, GK), jnp.int32),
          pltpu.SemaphoreType.DMA,
      ],
  )(feat_full, cat_full, grid)


def _tc_cat_body(lab_ref, cat_ref):
  lab = lab_ref[...]
  mx = jnp.max(lab, axis=1, keepdims=True)
  io = lax.broadcasted_iota(jnp.int32, lab.shape, 1)
  cat_ref[...] = jnp.min(jnp.where(lab == mx, io, 64), axis=1, keepdims=True)


@jax.jit
def _tc_cat(label_full):
  return pl.pallas_call(
      _tc_cat_body,
      grid=(N_FULL // 2000,),
      in_specs=[pl.BlockSpec((2000, 41), lambda i: (i, 0))],
      out_specs=pl.BlockSpec((2000, 1), lambda i: (i, 0)),
      out_shape=jax.ShapeDtypeStruct((N_FULL, 1), jnp.int32),
  )(label_full)


GC = 8
NG = NCHUNK // GC


ZR = 64


def _mul_rows(rows, w_v, j):
  @pl.loop(0, K // L)
  def _grp(gg):
    wvec = w_v[j, pl.ds(gg * L, L)]
    for i in range(L):
      wv = wvec[i]
      r = gg * L + i
      for cc in range(D // L):
        sl = pl.ds(cc * L, L)
        rows[r, sl] = rows[r, sl] * wv


def _spmm_body(x_hbm, src_hbm, dst_hbm, w_hbm, out_hbm,
               src_v, dst_v, w_v, rows_a, rows_b, zbuf, acc_sh,
               gsem_a, gsem_b, ssem_a, ssem_b):
  c = lax.axis_index("c")
  s = lax.axis_index("s")
  wid = s * NC + c
  bufs = (rows_a, rows_b)
  gsems = (gsem_a, gsem_b)
  ssems = (ssem_a, ssem_b)

  zero = jnp.zeros((L,), jnp.float32)

  @pl.loop(0, ZR)
  def _zrow(r):
    for cc in range(D // L):
      zbuf[r, pl.ds(cc * L, L)] = zero

  for t in range(RPT // ZR):
    pltpu.sync_copy(zbuf, acc_sh.at[pl.ds(s * RPT + t * ZR, ZR)])
  plsc.subcore_barrier()

  @pl.loop(0, NG)
  def _group(g):
    gsl = pl.ds(g * GC, GC)
    pltpu.sync_copy(src_hbm.at[wid, gsl], src_v)
    pltpu.sync_copy(dst_hbm.at[wid, gsl], dst_v)
    pltpu.sync_copy(w_hbm.at[wid, gsl], w_v)

    gd0 = pltpu.async_copy(x_hbm.at[src_v.at[0]], bufs[0], gsems[0])
    gds = [gd0, None]
    sds = [None, None]
    for j in range(GC):
      p = j % 2
      gds[p].wait()
      if j + 1 < GC:
        if sds[1 - p] is not None:
          sds[1 - p].wait()
        gds[1 - p] = pltpu.async_copy(
            x_hbm.at[src_v.at[j + 1]], bufs[1 - p], gsems[1 - p])
      _mul_rows(bufs[p], w_v, j)
      sds[p] = pltpu.async_copy(
          bufs[p], acc_sh.at[dst_v.at[j]], ssems[p], add=True)
    sds[0].wait()
    sds[1].wait()

  plsc.subcore_barrier()
  pltpu.sync_copy(acc_sh.at[pl.ds(s * RPT, RPT)],
                  out_hbm.at[c, pl.ds(s * RPT, RPT)])


@jax.jit
def _sc_spmm(x, src_g, dst_g, w_g):
  return pl.kernel(
      _spmm_body,
      out_type=jax.ShapeDtypeStruct((NC, R_PAD, D), jnp.float32),
      mesh=_vmesh,
      scratch_types=[
          pltpu.VMEM((GC, K), jnp.int32),
          pltpu.VMEM((GC, K), jnp.int32),
          pltpu.VMEM((GC, K), jnp.float32),
          pltpu.VMEM((K, D), jnp.float32),
          pltpu.VMEM((K, D), jnp.float32),
          pltpu.VMEM((ZR, D), jnp.float32),
          pltpu.VMEM_SHARED((R_PAD, D), jnp.float32),
          pltpu.SemaphoreType.DMA,
          pltpu.SemaphoreType.DMA,
          pltpu.SemaphoreType.DMA,
          pltpu.SemaphoreType.DMA,
      ],
  )(x, src_g, dst_g, w_g)


def _tc_a_body(x_ref, sp_ref, w0_ref, b0_ref, w1_ref, b1_ref, w1b_ref,
               h_ref, y_ref):
  x = x_ref[...]
  sp = sp_ref[0] + sp_ref[1]
  h0 = jnp.maximum(
      jnp.dot(x, w0_ref[...], preferred_element_type=jnp.float32)
      + b0_ref[...], 0.0)
  h1 = jnp.maximum(
      jnp.dot(sp, w1_ref[...], preferred_element_type=jnp.float32)
      + b1_ref[...], 0.0)
  h = jnp.concatenate([h0, h1], axis=1)
  h_ref[...] = h
  y_ref[...] = jnp.dot(h, w1b_ref[...], preferred_element_type=jnp.float32)


@jax.jit
def _tc_layer1(x, sp, w0, b0, w1, b1, w1b):
  return pl.pallas_call(
      _tc_a_body,
      grid=(GRID,),
      in_specs=[
          pl.BlockSpec((BR, D), lambda i: (i, 0)),
          pl.BlockSpec((NC, BR, D), lambda i: (0, i, 0)),
          pl.BlockSpec((D, D), lambda i: (0, 0)),
          pl.BlockSpec((1, D), lambda i: (0, 0)),
          pl.BlockSpec((D, D), lambda i: (0, 0)),
          pl.BlockSpec((1, D), lambda i: (0, 0)),
          pl.BlockSpec((2 * D, D), lambda i: (0, 0)),
      ],
      out_specs=[
          pl.BlockSpec((BR, 2 * D), lambda i: (i, 0)),
          pl.BlockSpec((BR, D), lambda i: (i, 0)),
      ],
      out_shape=[
          jax.ShapeDtypeStruct((R_PAD, 2 * D), jnp.float32),
          jax.ShapeDtypeStruct((R_PAD, D), jnp.float32),
      ],
  )(x, sp, w0, b0.reshape(1, D), w1, b1.reshape(1, D), w1b)


def _tc_b_body(h_ref, sp_ref, w0_ref, b0_ref, b1_ref,
               wc_ref, bc_ref, cat_ref, pred_ref, lab_ref):
  h = h_ref[...]
  h0 = jnp.maximum(
      jnp.dot(h, w0_ref[...], preferred_element_type=jnp.float32)
      + b0_ref[...], 0.0)
  h1 = jnp.maximum(sp_ref[0] + sp_ref[1] + b1_ref[...], 0.0)
  emb = jnp.concatenate([h0, h1], axis=1)
  n2 = jnp.sum(emb * emb, axis=1, keepdims=True)
  embn = emb / jnp.maximum(jnp.sqrt(n2), 1e-12)
  pred_ref[...] = (
      jnp.dot(embn, wc_ref[...], preferred_element_type=jnp.float32)
      + bc_ref[...])
  io = lax.broadcasted_iota(jnp.int32, (BR, C_CLS), 1)
  lab_ref[...] = (cat_ref[...] == io).astype(jnp.float32)


@jax.jit
def _tc_layer2(h, sp, w0, b0, b1, wc, bc, cat):
  return pl.pallas_call(
      _tc_b_body,
      grid=(GRID,),
      in_specs=[
          pl.BlockSpec((BR, 2 * D), lambda i: (i, 0)),
          pl.BlockSpec((NC, BR, D), lambda i: (0, i, 0)),
          pl.BlockSpec((2 * D, D), lambda i: (0, 0)),
          pl.BlockSpec((1, D), lambda i: (0, 0)),
          pl.BlockSpec((1, D), lambda i: (0, 0)),
          pl.BlockSpec((2 * D, C_CLS), lambda i: (0, 0)),
          pl.BlockSpec((1, C_CLS), lambda i: (0, 0)),
          pl.BlockSpec((BR, 1), lambda i: (i, 0)),
      ],
      out_specs=[
          pl.BlockSpec((BR, C_CLS), lambda i: (i, 0)),
          pl.BlockSpec((BR, C_CLS), lambda i: (i, 0)),
      ],
      out_shape=[
          jax.ShapeDtypeStruct((R_PAD, C_CLS), jnp.float32),
          jax.ShapeDtypeStruct((R_PAD, C_CLS), jnp.float32),
      ],
  )(h, sp, w0, b0.reshape(1, D), b1.reshape(1, D),
    wc, bc.reshape(1, C_CLS), cat)


def kernel(node_subgraph, edge_index, edge_weight, feat_full, label_full,
           W0a, b0a, W1a, b1a, W0b, b0b, W1b, b1b, Wc, bc):
  idx_pad = jnp.pad(node_subgraph, (0, R_PAD - N_SUB))
  src = jnp.pad(edge_index[0], (0, E_PAD - E)).reshape(NW, NCHUNK, K)
  dst = jnp.pad(edge_index[1], (0, E_PAD - E)).reshape(NW, NCHUNK, K)
  w = jnp.pad(edge_weight, (0, E_PAD - E)).reshape(NW, NCHUNK, K)

  cat_full = _tc_cat(label_full).reshape(N_FULL)
  feat_subg, cat_sub = _sc_gather(feat_full, cat_full, idx_pad)

  sp1 = _sc_spmm(feat_subg, src, dst, w)
  h, y = _tc_layer1(feat_subg, sp1, W0a, b0a, W1a, b1a, W1b)

  sp2 = _sc_spmm(y, src, dst, w)
  pred_pad, lab_pad = _tc_layer2(h, sp2, W0b, b0b, b1b, Wc, bc,
                                 cat_sub.reshape(R_PAD, 1))

  pred = pred_pad[:N_SUB]
  label_subg = lab_pad[:N_SUB]
  conv = cat_sub[:N_SUB]
  return pred, label_subg, conv
